# baseline (device time: 410716 ns/iter reference)
import jax
import jax.numpy as jnp
from jax import lax
from jax.experimental import pallas as pl
from jax.experimental.pallas import tpu as pltpu

CHUNKS = 8


def kernel(x):
    m, n2 = x.shape
    n = n2 // 2
    m2 = 2 * m
    rows = m // CHUNKS

    def body(x_ref, out_ref, sbuf, lbuf,
             send_sems, recv_sems, sfill_sems, lfill_sems, ldrain_sems):
        my_x = lax.axis_index("x")
        my_y = lax.axis_index("y")
        my_z = lax.axis_index("z")
        other = 1 - my_x

        barrier_sem = pltpu.get_barrier_semaphore()
        pl.semaphore_signal(
            barrier_sem, inc=1,
            device_id=(other, my_y, my_z),
            device_id_type=pl.DeviceIdType.MESH,
        )
        pl.semaphore_wait(barrier_sem, 1)

        def sfill(k):
            return pltpu.make_async_copy(
                x_ref.at[pl.ds(k * rows, rows), pl.ds(other * n, n)],
                sbuf.at[k],
                sfill_sems.at[k],
            )

        def rdma(k):
            return pltpu.make_async_remote_copy(
                src_ref=sbuf.at[k],
                dst_ref=out_ref.at[pl.ds(my_x * m + k * rows, rows), :],
                send_sem=send_sems.at[k],
                recv_sem=recv_sems.at[k],
                device_id=(other, my_y, my_z),
                device_id_type=pl.DeviceIdType.MESH,
            )

        def lfill(k):
            return pltpu.make_async_copy(
                x_ref.at[pl.ds(k * rows, rows), pl.ds(my_x * n, n)],
                lbuf.at[k % 2],
                lfill_sems.at[k % 2],
            )

        def ldrain(k):
            return pltpu.make_async_copy(
                lbuf.at[k % 2],
                out_ref.at[pl.ds(my_x * m + k * rows, rows), :],
                ldrain_sems.at[k % 2],
            )

        sfill(0).start()
        sfill(1).start()
        lfill(0).start()

        rdmas = []
        for k in range(CHUNKS):
            sfill(k).wait()
            r = rdma(k)
            r.start()
            rdmas.append(r)
            if k + 2 < CHUNKS:
                sfill(k + 2).start()
            lfill(k).wait()
            if k >= 1:
                ldrain(k - 1).wait()
            if k + 1 < CHUNKS:
                lfill(k + 1).start()
            ldrain(k).start()

        ldrain(CHUNKS - 1).wait()
        for r in rdmas:
            r.wait()

    return pl.pallas_call(
        body,
        out_shape=jax.ShapeDtypeStruct((m2, n), x.dtype),
        in_specs=[pl.BlockSpec(memory_space=pl.ANY)],
        out_specs=pl.BlockSpec(memory_space=pl.ANY),
        scratch_shapes=[
            pltpu.VMEM((CHUNKS, rows, n), x.dtype),
            pltpu.VMEM((2, rows, n), x.dtype),
            pltpu.SemaphoreType.DMA((CHUNKS,)),
            pltpu.SemaphoreType.DMA((CHUNKS,)),
            pltpu.SemaphoreType.DMA((CHUNKS,)),
            pltpu.SemaphoreType.DMA((2,)),
            pltpu.SemaphoreType.DMA((2,)),
        ],
        compiler_params=pltpu.CompilerParams(
            collective_id=0,
            vmem_limit_bytes=48 * 1024 * 1024,
        ),
    )(x)


# device time: 408922 ns/iter; 1.0044x vs baseline; 1.0044x over previous
import jax
import jax.numpy as jnp
from jax import lax
from jax.experimental import pallas as pl
from jax.experimental.pallas import tpu as pltpu

CHUNKS = 8


def kernel(x):
    m, n2 = x.shape
    n = n2 // 2
    m2 = 2 * m
    rows = m // CHUNKS

    def body(x_ref, out_ref, stage, send_sems, recv_sems, in_sems, out_sems):
        my_x = lax.axis_index("x")
        my_y = lax.axis_index("y")
        my_z = lax.axis_index("z")
        other = 1 - my_x

        barrier_sem = pltpu.get_barrier_semaphore()
        pl.semaphore_signal(
            barrier_sem, inc=1,
            device_id=(other, my_y, my_z),
            device_id_type=pl.DeviceIdType.MESH,
        )
        pl.semaphore_wait(barrier_sem, 1)

        rdmas = []
        for k in range(CHUNKS):
            rdma = pltpu.make_async_remote_copy(
                src_ref=x_ref.at[pl.ds(k * rows, rows), pl.ds(other * n, n)],
                dst_ref=out_ref.at[pl.ds(my_x * m + k * rows, rows), :],
                send_sem=send_sems.at[k],
                recv_sem=recv_sems.at[k],
                device_id=(other, my_y, my_z),
                device_id_type=pl.DeviceIdType.MESH,
            )
            rdma.start()
            rdmas.append(rdma)

        def copy_in(k):
            return pltpu.make_async_copy(
                x_ref.at[pl.ds(k * rows, rows), pl.ds(my_x * n, n)],
                stage.at[k % 2],
                in_sems.at[k % 2],
            )

        def copy_out(k):
            return pltpu.make_async_copy(
                stage.at[k % 2],
                out_ref.at[pl.ds(my_x * m + k * rows, rows), :],
                out_sems.at[k % 2],
            )

        copy_in(0).start()
        for k in range(CHUNKS):
            copy_in(k).wait()
            if k >= 1:
                copy_out(k - 1).wait()
            if k + 1 < CHUNKS:
                copy_in(k + 1).start()
            copy_out(k).start()
        copy_out(CHUNKS - 1).wait()

        for rdma in rdmas:
            rdma.wait()

    return pl.pallas_call(
        body,
        out_shape=jax.ShapeDtypeStruct((m2, n), x.dtype),
        in_specs=[pl.BlockSpec(memory_space=pl.ANY)],
        out_specs=pl.BlockSpec(memory_space=pl.ANY),
        scratch_shapes=[
            pltpu.VMEM((2, rows, n), x.dtype),
            pltpu.SemaphoreType.DMA((CHUNKS,)),
            pltpu.SemaphoreType.DMA((CHUNKS,)),
            pltpu.SemaphoreType.DMA((2,)),
            pltpu.SemaphoreType.DMA((2,)),
        ],
        compiler_params=pltpu.CompilerParams(collective_id=0),
    )(x)
